# baseline (speedup 1.0000x reference)
import jax
import jax.numpy as jnp
from jax import lax
from jax.experimental import pallas as pl
from jax.experimental.pallas import tpu as pltpu
from jax.experimental.pallas import tpu_sc as plsc

B, T, C = 64, 2048, 15
BLANK = C - 1
MAXLEN = 20
OUTW = 24
NC, NS, L = 1, 16, 16
ROWS_PER_W = B // (NC * NS)
CT = 128
NCHUNKS = T // CT
SUBCH = CT // L
NROWSEG = B // 8 * NCHUNKS * 8


def _decode_body(y_ref, out_ref, *scr):
    bufs = scr[0:ROWS_PER_W]
    tmp = scr[ROWS_PER_W]
    obufs = scr[ROWS_PER_W + 1:2 * ROWS_PER_W + 1]
    sems = scr[2 * ROWS_PER_W + 1:3 * ROWS_PER_W + 1]
    osem = scr[3 * ROWS_PER_W + 1]
    cid = lax.axis_index("c")
    sid = lax.axis_index("s")
    wid = sid * NC + cid
    lanes = lax.broadcasted_iota(jnp.int32, (L,), 0)
    neg1 = jnp.full((L,), -1, jnp.int32)
    cvec = jnp.minimum(lanes, C - 1)

    rows = [wid * ROWS_PER_W + r for r in range(ROWS_PER_W)]
    seg_bases = [cvec * NROWSEG + (row // 8) * (NCHUNKS * 8) + row % 8
                 for row in rows]
    descs = [pltpu.async_copy(y_ref.at[seg_bases[r]], bufs[r], sems[r])
             for r in range(ROWS_PER_W)]
    for ob in obufs:
        ob[pl.ds(0, L)] = neg1
        ob[pl.ds(OUTW - L, L)] = neg1

    out_descs = []
    for r in range(ROWS_PER_W):
        buf = bufs[r]
        obuf = obufs[r]

        def sub_body(s, carry, buf=buf, obuf=obuf):
            ntok, prevv = carry
            t0 = s * L
            bval = buf[0, pl.ds(t0, L)]
            btok = jnp.zeros((L,), jnp.int32)
            for c in range(1, C):
                v = buf[c, pl.ds(t0, L)]
                upd = v > bval
                bval = jnp.where(upd, v, bval)
                btok = jnp.where(upd, c, btok)
            plsc.store_scatter(tmp, [lanes], prevv, mask=lanes == 0)
            plsc.store_scatter(tmp, [lanes + 1], btok)
            pvec = plsc.load_gather(tmp, [lanes])
            keep = (btok != pvec) & (btok != BLANK)
            cum = plsc.cumsum(jnp.where(keep, 1, 0).astype(jnp.int32))
            rank = ntok + cum - 1
            wmask = keep & (rank < MAXLEN)
            plsc.store_scatter(obuf, [jnp.minimum(rank, MAXLEN)], btok,
                               mask=wmask)
            ntok = ntok + jnp.max(cum)
            prevv = plsc.load_gather(tmp, [jnp.full((L,), L, jnp.int32)])
            return ntok, prevv

        descs[r].wait()
        ntok, prevv = lax.fori_loop(0, SUBCH, sub_body,
                                    (jnp.int32(0), neg1))

        def chunk_cond(carry):
            ct, ntok, _ = carry
            return (ct < NCHUNKS) & (ntok < MAXLEN)

        def chunk_body(carry, r=r, buf=buf, sub_body=sub_body):
            ct, ntok, prevv = carry
            pltpu.async_copy(y_ref.at[seg_bases[r] + ct * 8], buf,
                             sems[r]).wait()
            ntok, prevv = lax.fori_loop(0, SUBCH, sub_body, (ntok, prevv))
            return ct + 1, ntok, prevv

        lax.while_loop(chunk_cond, chunk_body, (jnp.int32(1), ntok, prevv))
        out_descs.append(
            pltpu.async_copy(obufs[r], out_ref.at[rows[r]], osem))
    for d in out_descs:
        d.wait()


@jax.jit
def kernel(y_pred):
    y_view = (
        y_pred.transpose(2, 0, 1)
        .reshape(C, B // 8, 8, T // CT, CT)
        .transpose(0, 1, 3, 2, 4)
        .reshape(C * NROWSEG, CT)
    )
    mesh = plsc.VectorSubcoreMesh(core_axis_name="c", subcore_axis_name="s",
                                  num_cores=NC, num_subcores=NS)
    out = pl.kernel(
        _decode_body,
        out_type=jax.ShapeDtypeStruct((B, OUTW), jnp.int32),
        mesh=mesh,
        compiler_params=pltpu.CompilerParams(needs_layout_passes=False),
        scratch_types=(
            [pltpu.VMEM((L, CT), jnp.float32)] * ROWS_PER_W
            + [pltpu.VMEM((L + 1,), jnp.int32)]
            + [pltpu.VMEM((OUTW,), jnp.int32)] * ROWS_PER_W
            + [pltpu.SemaphoreType.DMA] * (ROWS_PER_W + 1)
        ),
    )(y_view)
    return out[:, :MAXLEN]


# speedup vs baseline: 1.0404x; 1.0404x over previous
import jax
import jax.numpy as jnp
from jax import lax
from jax.experimental import pallas as pl
from jax.experimental.pallas import tpu as pltpu
from jax.experimental.pallas import tpu_sc as plsc

B, T, C = 64, 2048, 15
BLANK = C - 1
MAXLEN = 20
OUTW = 24
NC, NS, L = 2, 16, 16
ROWS_PER_W = B // (NC * NS)
CT = 128
NCHUNKS = T // CT
SUBCH = CT // L
NROWSEG = B // 8 * NCHUNKS * 8


def _decode_body(y_ref, out_ref, *scr):
    bufs = scr[0:ROWS_PER_W]
    tmp = scr[ROWS_PER_W]
    obufs = scr[ROWS_PER_W + 1:2 * ROWS_PER_W + 1]
    sems = scr[2 * ROWS_PER_W + 1:3 * ROWS_PER_W + 1]
    osem = scr[3 * ROWS_PER_W + 1]
    cid = lax.axis_index("c")
    sid = lax.axis_index("s")
    wid = sid * NC + cid
    lanes = lax.broadcasted_iota(jnp.int32, (L,), 0)
    neg1 = jnp.full((L,), -1, jnp.int32)
    cvec = jnp.minimum(lanes, C - 1)

    rows = [wid * ROWS_PER_W + r for r in range(ROWS_PER_W)]
    seg_bases = [cvec * NROWSEG + (row // 8) * (NCHUNKS * 8) + row % 8
                 for row in rows]
    descs = [pltpu.async_copy(y_ref.at[seg_bases[r]], bufs[r], sems[r])
             for r in range(ROWS_PER_W)]
    for ob in obufs:
        ob[pl.ds(0, L)] = neg1
        ob[pl.ds(OUTW - L, L)] = neg1

    out_descs = []
    for r in range(ROWS_PER_W):
        buf = bufs[r]
        obuf = obufs[r]

        def sub_body(s, carry, buf=buf, obuf=obuf):
            ntok, prevv = carry
            t0 = s * L
            bval = buf[0, pl.ds(t0, L)]
            btok = jnp.zeros((L,), jnp.int32)
            for c in range(1, C):
                v = buf[c, pl.ds(t0, L)]
                upd = v > bval
                bval = jnp.where(upd, v, bval)
                btok = jnp.where(upd, c, btok)
            plsc.store_scatter(tmp, [lanes], prevv, mask=lanes == 0)
            plsc.store_scatter(tmp, [lanes + 1], btok)
            pvec = plsc.load_gather(tmp, [lanes])
            keep = (btok != pvec) & (btok != BLANK)
            cum = plsc.cumsum(jnp.where(keep, 1, 0).astype(jnp.int32))
            rank = ntok + cum - 1
            wmask = keep & (rank < MAXLEN)
            plsc.store_scatter(obuf, [jnp.minimum(rank, MAXLEN)], btok,
                               mask=wmask)
            ntok = ntok + jnp.max(cum)
            prevv = plsc.load_gather(tmp, [jnp.full((L,), L, jnp.int32)])
            return ntok, prevv

        descs[r].wait()
        ntok, prevv = lax.fori_loop(0, SUBCH, sub_body,
                                    (jnp.int32(0), neg1))

        def chunk_cond(carry):
            ct, ntok, _ = carry
            return (ct < NCHUNKS) & (ntok < MAXLEN)

        def chunk_body(carry, r=r, buf=buf, sub_body=sub_body):
            ct, ntok, prevv = carry
            pltpu.async_copy(y_ref.at[seg_bases[r] + ct * 8], buf,
                             sems[r]).wait()
            ntok, prevv = lax.fori_loop(0, SUBCH, sub_body, (ntok, prevv))
            return ct + 1, ntok, prevv

        lax.while_loop(chunk_cond, chunk_body, (jnp.int32(1), ntok, prevv))
        out_descs.append(
            pltpu.async_copy(obufs[r], out_ref.at[rows[r]], osem))
    for d in out_descs:
        d.wait()


@jax.jit
def kernel(y_pred):
    y_view = (
        y_pred.transpose(2, 0, 1)
        .reshape(C, B // 8, 8, T // CT, CT)
        .transpose(0, 1, 3, 2, 4)
        .reshape(C * NROWSEG, CT)
    )
    mesh = plsc.VectorSubcoreMesh(core_axis_name="c", subcore_axis_name="s",
                                  num_cores=NC, num_subcores=NS)
    out = pl.kernel(
        _decode_body,
        out_type=jax.ShapeDtypeStruct((B, OUTW), jnp.int32),
        mesh=mesh,
        compiler_params=pltpu.CompilerParams(needs_layout_passes=False),
        scratch_types=(
            [pltpu.VMEM((L, CT), jnp.float32)] * ROWS_PER_W
            + [pltpu.VMEM((L + 1,), jnp.int32)]
            + [pltpu.VMEM((OUTW,), jnp.int32)] * ROWS_PER_W
            + [pltpu.SemaphoreType.DMA] * (ROWS_PER_W + 1)
        ),
    )(y_view)
    return out[:, :MAXLEN]


# speedup vs baseline: 1.1040x; 1.0611x over previous
import jax
import jax.numpy as jnp
from jax import lax
from jax.experimental import pallas as pl
from jax.experimental.pallas import tpu as pltpu
from jax.experimental.pallas import tpu_sc as plsc

B, T, C = 64, 2048, 15
BLANK = C - 1
MAXLEN = 20
OUTW = 24
NC, NS, L = 2, 16, 16
ROWS_PER_W = B // (NC * NS)
CT = 128
NCHUNKS = T // CT
SUBCH = CT // L
NROWSEG = B // 8 * NCHUNKS * 8


def _decode_body(y_ref, out_ref, *scr):
    bufs = scr[0:ROWS_PER_W]
    tmp = scr[ROWS_PER_W]
    obufs = scr[ROWS_PER_W + 1:2 * ROWS_PER_W + 1]
    sems = scr[2 * ROWS_PER_W + 1:3 * ROWS_PER_W + 1]
    osem = scr[3 * ROWS_PER_W + 1]
    cid = lax.axis_index("c")
    sid = lax.axis_index("s")
    wid = sid * NC + cid
    lanes = lax.broadcasted_iota(jnp.int32, (L,), 0)
    neg1 = jnp.full((L,), -1, jnp.int32)
    cvec = jnp.minimum(lanes, C - 1)

    rows = [wid * ROWS_PER_W + r for r in range(ROWS_PER_W)]
    seg_bases = [cvec * NROWSEG + (row // 8) * (NCHUNKS * 8) + row % 8
                 for row in rows]
    descs = [pltpu.async_copy(y_ref.at[seg_bases[r]], bufs[r], sems[r])
             for r in range(ROWS_PER_W)]
    for ob in obufs:
        ob[pl.ds(0, L)] = neg1
        ob[pl.ds(OUTW - L, L)] = neg1

    out_descs = []
    for r in range(ROWS_PER_W):
        buf = bufs[r]
        obuf = obufs[r]

        def sub_body(s, carry, buf=buf, obuf=obuf):
            ntok, prevv = carry
            t0 = s * L
            bval = buf[0, pl.ds(t0, L)]
            btok = jnp.zeros((L,), jnp.int32)
            for c in range(1, C):
                v = buf[c, pl.ds(t0, L)]
                upd = v > bval
                bval = jnp.where(upd, v, bval)
                btok = jnp.where(upd, c, btok)
            plsc.store_scatter(tmp, [lanes], prevv, mask=lanes == 0)
            plsc.store_scatter(tmp, [lanes + 1], btok)
            pvec = plsc.load_gather(tmp, [lanes])
            keep = (btok != pvec) & (btok != BLANK)
            cum = plsc.cumsum(jnp.where(keep, 1, 0).astype(jnp.int32))
            rank = ntok + cum - 1
            wmask = keep & (rank < MAXLEN)
            plsc.store_scatter(obuf, [jnp.minimum(rank, MAXLEN)], btok,
                               mask=wmask)
            ntok = ntok + jnp.max(cum)
            prevv = plsc.load_gather(tmp, [jnp.full((L,), L, jnp.int32)])
            return ntok, prevv

        def sub_cond(carry):
            ss, ntok, _ = carry
            return (ss < SUBCH) & (ntok < MAXLEN)

        def sub_while_body(carry, sub_body=sub_body):
            ss, ntok, prevv = carry
            ntok, prevv = sub_body(ss, (ntok, prevv))
            return ss + 1, ntok, prevv

        descs[r].wait()
        _, ntok, prevv = lax.while_loop(sub_cond, sub_while_body,
                                        (jnp.int32(0), jnp.int32(0), neg1))

        def chunk_cond(carry):
            ct, ntok, _ = carry
            return (ct < NCHUNKS) & (ntok < MAXLEN)

        def chunk_body(carry, r=r, buf=buf, sub_cond=sub_cond,
                       sub_while_body=sub_while_body):
            ct, ntok, prevv = carry
            pltpu.async_copy(y_ref.at[seg_bases[r] + ct * 8], buf,
                             sems[r]).wait()
            _, ntok, prevv = lax.while_loop(sub_cond, sub_while_body,
                                            (jnp.int32(0), ntok, prevv))
            return ct + 1, ntok, prevv

        lax.while_loop(chunk_cond, chunk_body, (jnp.int32(1), ntok, prevv))
        out_descs.append(
            pltpu.async_copy(obufs[r], out_ref.at[rows[r]], osem))
    for d in out_descs:
        d.wait()


@jax.jit
def kernel(y_pred):
    y_view = (
        y_pred.transpose(2, 0, 1)
        .reshape(C, B // 8, 8, T // CT, CT)
        .transpose(0, 1, 3, 2, 4)
        .reshape(C * NROWSEG, CT)
    )
    mesh = plsc.VectorSubcoreMesh(core_axis_name="c", subcore_axis_name="s",
                                  num_cores=NC, num_subcores=NS)
    out = pl.kernel(
        _decode_body,
        out_type=jax.ShapeDtypeStruct((B, OUTW), jnp.int32),
        mesh=mesh,
        compiler_params=pltpu.CompilerParams(needs_layout_passes=False),
        scratch_types=(
            [pltpu.VMEM((L, CT), jnp.float32)] * ROWS_PER_W
            + [pltpu.VMEM((L + 1,), jnp.int32)]
            + [pltpu.VMEM((OUTW,), jnp.int32)] * ROWS_PER_W
            + [pltpu.SemaphoreType.DMA] * (ROWS_PER_W + 1)
        ),
    )(y_view)
    return out[:, :MAXLEN]


# speedup vs baseline: 1.1063x; 1.0021x over previous
import jax
import jax.numpy as jnp
from jax import lax
from jax.experimental import pallas as pl
from jax.experimental.pallas import tpu as pltpu
from jax.experimental.pallas import tpu_sc as plsc

B, T, C = 64, 2048, 15
BLANK = C - 1
MAXLEN = 20
OUTW = 24
NC, NS, L = 2, 16, 16
ROWS_PER_W = B // (NC * NS)
CT = 128
NCHUNKS = T // CT
SUBCH = CT // L
NROWSEG = B // 8 * NCHUNKS * 8


def _decode_body(y_ref, out_ref, *scr):
    bufs = scr[0:ROWS_PER_W]
    tmp = scr[ROWS_PER_W]
    obufs = scr[ROWS_PER_W + 1:2 * ROWS_PER_W + 1]
    sems = scr[2 * ROWS_PER_W + 1:3 * ROWS_PER_W + 1]
    osem = scr[3 * ROWS_PER_W + 1]
    cid = lax.axis_index("c")
    sid = lax.axis_index("s")
    wid = sid * NC + cid
    lanes = lax.broadcasted_iota(jnp.int32, (L,), 0)
    neg1 = jnp.full((L,), -1, jnp.int32)
    cvec = jnp.minimum(lanes, C - 1)

    rows = [wid * ROWS_PER_W + r for r in range(ROWS_PER_W)]
    seg_bases = [cvec * NROWSEG + (row // 8) * (NCHUNKS * 8) + row % 8
                 for row in rows]
    descs = [pltpu.async_copy(y_ref.at[seg_bases[r]], bufs[r], sems[r])
             for r in range(ROWS_PER_W)]
    for ob in obufs:
        ob[pl.ds(0, L)] = neg1
        ob[pl.ds(OUTW - L, L)] = neg1

    out_descs = []
    for r in range(ROWS_PER_W):
        buf = bufs[r]
        obuf = obufs[r]

        def sub_body(s, carry, buf=buf, obuf=obuf):
            ntok, prevv = carry
            t0 = s * L
            bval = buf[0, pl.ds(t0, L)]
            btok = jnp.zeros((L,), jnp.int32)
            for c in range(1, C):
                v = buf[c, pl.ds(t0, L)]
                upd = v > bval
                bval = jnp.where(upd, v, bval)
                btok = jnp.where(upd, c, btok)
            shifted = jnp.take_along_axis(btok, jnp.maximum(lanes - 1, 0),
                                          axis=0)
            pvec = jnp.where(lanes == 0, prevv, shifted)
            keep = (btok != pvec) & (btok != BLANK)
            cum = plsc.cumsum(jnp.where(keep, 1, 0).astype(jnp.int32))
            rank = ntok + cum - 1
            wmask = keep & (rank < MAXLEN)
            plsc.store_scatter(obuf, [jnp.minimum(rank, MAXLEN)], btok,
                               mask=wmask)
            ntok = ntok + jnp.squeeze(lax.slice(cum, (L - 1,), (L,)))
            prevv = jnp.take_along_axis(
                btok, jnp.full((L,), L - 1, jnp.int32), axis=0)
            return ntok, prevv

        def sub_cond(carry):
            ss, ntok, _ = carry
            return (ss < SUBCH) & (ntok < MAXLEN)

        def sub_while_body(carry, sub_body=sub_body):
            ss, ntok, prevv = carry
            ntok, prevv = sub_body(ss, (ntok, prevv))
            return ss + 1, ntok, prevv

        descs[r].wait()
        _, ntok, prevv = lax.while_loop(sub_cond, sub_while_body,
                                        (jnp.int32(0), jnp.int32(0), neg1))

        def chunk_cond(carry):
            ct, ntok, _ = carry
            return (ct < NCHUNKS) & (ntok < MAXLEN)

        def chunk_body(carry, r=r, buf=buf, sub_cond=sub_cond,
                       sub_while_body=sub_while_body):
            ct, ntok, prevv = carry
            pltpu.async_copy(y_ref.at[seg_bases[r] + ct * 8], buf,
                             sems[r]).wait()
            _, ntok, prevv = lax.while_loop(sub_cond, sub_while_body,
                                            (jnp.int32(0), ntok, prevv))
            return ct + 1, ntok, prevv

        lax.while_loop(chunk_cond, chunk_body, (jnp.int32(1), ntok, prevv))
        out_descs.append(
            pltpu.async_copy(obufs[r], out_ref.at[rows[r]], osem))
    for d in out_descs:
        d.wait()


@jax.jit
def kernel(y_pred):
    y_view = (
        y_pred.transpose(2, 0, 1)
        .reshape(C, B // 8, 8, T // CT, CT)
        .transpose(0, 1, 3, 2, 4)
        .reshape(C * NROWSEG, CT)
    )
    mesh = plsc.VectorSubcoreMesh(core_axis_name="c", subcore_axis_name="s",
                                  num_cores=NC, num_subcores=NS)
    out = pl.kernel(
        _decode_body,
        out_type=jax.ShapeDtypeStruct((B, OUTW), jnp.int32),
        mesh=mesh,
        compiler_params=pltpu.CompilerParams(needs_layout_passes=False),
        scratch_types=(
            [pltpu.VMEM((L, CT), jnp.float32)] * ROWS_PER_W
            + [pltpu.VMEM((L + 1,), jnp.int32)]
            + [pltpu.VMEM((OUTW,), jnp.int32)] * ROWS_PER_W
            + [pltpu.SemaphoreType.DMA] * (ROWS_PER_W + 1)
        ),
    )(y_view)
    return out[:, :MAXLEN]
